# baseline (speedup 1.0000x reference)
import functools
import jax
import jax.numpy as jnp
from jax import lax
from jax.experimental import pallas as pl
from jax.experimental.pallas import tpu as pltpu
from jax.experimental.pallas import tpu_sc as plsc

_N = 10000
_NPAD = 10112
_E = 320000
_CHUNK = 128
_CW = 128
_NCH_A = _E // 16 // _CHUNK + 1
_NCH_B = _E // 2 // 16 // _CHUNK + 1
_HALF = 5056
_ACC_ROWS = 5120
_ACC_SUB = _ACC_ROWS // 16
_TRASH = 5118


def _make_sc_prop(nch, shared_table):
    mesh = plsc.VectorSubcoreMesh(core_axis_name="c", subcore_axis_name="s")

    @functools.partial(
        pl.kernel,
        mesh=mesh,
        out_type=jax.ShapeDtypeStruct((2, _ACC_ROWS, _CW), jnp.float32),
        scratch_types=[
            pltpu.VMEM((nch, _CHUNK), jnp.int32),
            pltpu.VMEM((nch, _CHUNK), jnp.int32),
            pltpu.VMEM((2, _CHUNK, _CW), jnp.float32),
            pltpu.VMEM_SHARED((_ACC_ROWS, _CW), jnp.float32),
            pltpu.SemaphoreType.DMA,
        ],
    )
    def sc_prop(zt_hbm, rowp_hbm, col_hbm, zeros_hbm, out_hbm,
                rowv, colv, gbuf, acc, sem):
        cid = lax.axis_index("c")
        sid = lax.axis_index("s")
        r0 = sid * _ACC_SUB
        pltpu.sync_copy(zeros_hbm.at[pl.ds(r0, _ACC_SUB)],
                        acc.at[pl.ds(r0, _ACC_SUB)])
        pltpu.sync_copy(rowp_hbm.at[cid].at[sid], rowv)
        pltpu.sync_copy(col_hbm.at[cid].at[sid], colv)
        plsc.subcore_barrier()

        zt2d = zt_hbm.at[0] if shared_table else zt_hbm.at[cid]
        pltpu.async_copy(zt2d.at[rowv.at[0]], gbuf.at[0], sem)

        def body(j, carry):
            @pl.when(j + 1 < nch)
            def _():
                pltpu.async_copy(zt2d.at[rowv.at[j + 1]],
                                 gbuf.at[(j + 1) % 2], sem)
            pltpu.make_async_copy(zt2d.at[rowv.at[j]],
                                  gbuf.at[j % 2], sem).wait()
            pltpu.sync_copy(gbuf.at[j % 2], acc.at[colv.at[j]], add=True)
            return carry

        lax.fori_loop(0, nch, body, 0)
        plsc.subcore_barrier()
        pltpu.sync_copy(acc.at[pl.ds(r0, _ACC_SUB)],
                        out_hbm.at[cid].at[pl.ds(r0, _ACC_SUB)])

    return sc_prop


_PROPS = {}


def _get_prop(flavor):
    if flavor not in _PROPS:
        if flavor == "A":
            _PROPS[flavor] = _make_sc_prop(_NCH_A, shared_table=False)
        else:
            _PROPS[flavor] = _make_sc_prop(_NCH_B, shared_table=True)
    return _PROPS[flavor]


def _chunked(rowp, colp, n_parts, nch):
    per_sub = nch * _CHUNK
    total = n_parts * 16 * per_sub
    pad = total - rowp.shape[0]
    rowp = jnp.concatenate([rowp, jnp.full((pad,), _N, jnp.int32)])
    colp = jnp.concatenate([colp, jnp.full((pad,), _TRASH, jnp.int32)])
    shape = (n_parts, 16, nch, _CHUNK)
    return rowp.reshape(shape), colp.reshape(shape)


def _halve(colp, h):
    base = h * _HALF
    inh = (colp >= base) & (colp < base + _HALF)
    return jnp.where(inh, colp - base, _TRASH).astype(jnp.int32)


def _split_cores(rowp, colp, nch):
    half = rowp.shape[0] // 2
    r0, c0 = _chunked(rowp[:half], colp[:half], 1, nch)
    r1, c1 = _chunked(rowp[half:], colp[half:], 1, nch)
    return (jnp.concatenate([r0, r1], axis=0),
            jnp.concatenate([c0, c1], axis=0))


def _edge_prep(row, col):
    rowp = jnp.where(row == col, _N, row).astype(jnp.int32)
    colp = col.astype(jnp.int32)
    ea, eb, ed = [], [], []
    for h in (0, 1):
        ch = _halve(colp, h)
        ra, ca = _chunked(rowp, ch, 1, _NCH_A)
        ea.append((jnp.concatenate([ra, ra], axis=0),
                   jnp.concatenate([ca, ca], axis=0)))
        eb.append(_split_cores(rowp, ch, _NCH_B))
        ed.append(_split_cores(rowp, _halve(row.astype(jnp.int32), h),
                               _NCH_B))
    return ea, eb, ed


def _run_halves(flavor, zts, eh, zeros, combine):
    outs = []
    for h in (0, 1):
        acc = _get_prop(flavor)(zts, eh[h][0], eh[h][1], zeros)
        outs.append(combine(acc)[:_HALF])
    return jnp.concatenate(outs, axis=0)


def _dis_vec(ed, zeros, n):
    ones = jnp.pad(jnp.ones((n, 1), jnp.float32),
                   ((0, _NPAD - n), (0, _CW - 1)))[None]
    deg = _run_halves("B", ones, ed, zeros, lambda a: a[0] + a[1])[:, 0]
    dis = jnp.where(deg > 0, 1.0 / jnp.sqrt(jnp.where(deg > 0, deg, 1.0)), 0.0)
    return dis[:, None]


def _sc_prop(zt, ea, eb, zeros):
    c = zt.shape[1]
    if c == 2 * _CW:
        zts = jnp.stack([zt[:, :_CW], zt[:, _CW:]], axis=0)
        return _run_halves(
            "A", zts, ea, zeros,
            lambda a: jnp.concatenate([a[0], a[1]], axis=1))
    ztp = jnp.pad(zt, ((0, 0), (0, _CW - c)))[None]
    return _run_halves("B", ztp, eb, zeros,
                       lambda a: a[0] + a[1])[:, :c]


def _cheb_conv(h, dis, ea, eb, zeros, W, b):
    Tx0 = h
    out = Tx0 @ W[0]
    acc = _sc_prop(dis * Tx0, ea, eb, zeros)
    Tx1 = -dis * acc
    out = out + Tx1 @ W[1]
    for k in range(2, W.shape[0]):
        acc = _sc_prop(dis * Tx1, ea, eb, zeros)
        Tx2 = -2.0 * dis * acc - Tx0
        out = out + Tx2 @ W[k]
        Tx0, Tx1 = Tx1, Tx2
    return out + b


def _branch(x, row, col, n, params):
    ea, eb, ed = _edge_prep(row, col)
    zeros = jnp.zeros((_ACC_ROWS, _CW), jnp.float32)
    dis = _dis_vec(ed, zeros, n)
    h = jnp.pad(x, ((0, _NPAD - n), (0, 0)))
    for i, (W, b) in enumerate(params):
        h = _cheb_conv(h, dis, ea, eb, zeros, W, b)
        if i < len(params) - 1:
            h = jax.nn.relu(h)
    return h[:n]


def kernel(x, edge_index, params1, params2, params3):
    n1 = _N
    n3 = edge_index.shape[1] // 3
    e1 = edge_index[:, 0:n3]
    e2 = edge_index[:, n3:2 * n3]
    e3 = edge_index[:, 2 * n3:]
    x1 = x[0:2 * n1:2, :]
    x2 = x[1:2 * n1:2, :]
    x3 = x[2 * n1:, :]
    o1 = _branch(x1, e1[0], e1[1], n1, params1)
    o2 = _branch(x2, e2[0], e2[1], n1, params2)
    o3 = _branch(x3, e3[0], e3[1], x3.shape[0], params3)
    uv = jnp.stack([o1, o2], axis=1).reshape(2 * n1, o1.shape[1])
    return jnp.concatenate([uv, o3], axis=0)


# speedup vs baseline: 1.0003x; 1.0003x over previous
import functools
import jax
import jax.numpy as jnp
from jax import lax
from jax.experimental import pallas as pl
from jax.experimental.pallas import tpu as pltpu
from jax.experimental.pallas import tpu_sc as plsc

_N = 10000
_NPAD = 10112
_E = 320000
_CHUNK = 128
_CW = 128
_NCH_A = _E // 16 // _CHUNK + 1
_NCH_B = _E // 2 // 16 // _CHUNK + 1
_HALF = 5056
_ACC_ROWS = 5120
_ACC_SUB = _ACC_ROWS // 16
_TRASH = 5118


def _make_sc_prop(nch, shared_table):
    mesh = plsc.VectorSubcoreMesh(core_axis_name="c", subcore_axis_name="s")

    @functools.partial(
        pl.kernel,
        mesh=mesh,
        out_type=jax.ShapeDtypeStruct((2, _ACC_ROWS, _CW), jnp.float32),
        scratch_types=[
            pltpu.VMEM((nch, _CHUNK), jnp.int32),
            pltpu.VMEM((nch, _CHUNK), jnp.int32),
            pltpu.VMEM((2, _CHUNK, _CW), jnp.float32),
            pltpu.VMEM_SHARED((_ACC_ROWS, _CW), jnp.float32),
            pltpu.SemaphoreType.DMA,
        ],
    )
    def sc_prop(zt_hbm, rowp_hbm, col_hbm, zeros_hbm, out_hbm,
                rowv, colv, gbuf, acc, gsem):
        cid = lax.axis_index("c")
        sid = lax.axis_index("s")
        r0 = sid * _ACC_SUB
        pltpu.sync_copy(zeros_hbm.at[pl.ds(r0, _ACC_SUB)],
                        acc.at[pl.ds(r0, _ACC_SUB)])
        pltpu.sync_copy(rowp_hbm.at[cid].at[sid], rowv)
        pltpu.sync_copy(col_hbm.at[cid].at[sid], colv)
        plsc.subcore_barrier()

        zt2d = zt_hbm.at[0] if shared_table else zt_hbm.at[cid]
        pltpu.async_copy(zt2d.at[rowv.at[0]], gbuf.at[0], gsem)

        def body(j, carry):
            @pl.when(j + 1 < nch)
            def _():
                pltpu.async_copy(zt2d.at[rowv.at[j + 1]],
                                 gbuf.at[(j + 1) % 2], gsem)
            pltpu.make_async_copy(zt2d.at[rowv.at[j]],
                                  gbuf.at[j % 2], gsem).wait()
            pltpu.sync_copy(gbuf.at[j % 2], acc.at[colv.at[j]], add=True)
            return carry

        lax.fori_loop(0, nch, body, 0)
        plsc.subcore_barrier()
        pltpu.sync_copy(acc.at[pl.ds(r0, _ACC_SUB)],
                        out_hbm.at[cid].at[pl.ds(r0, _ACC_SUB)])

    return sc_prop


_PROPS = {}


def _get_prop(flavor):
    if flavor not in _PROPS:
        if flavor == "A":
            _PROPS[flavor] = _make_sc_prop(_NCH_A, shared_table=False)
        else:
            _PROPS[flavor] = _make_sc_prop(_NCH_B, shared_table=True)
    return _PROPS[flavor]


def _chunked(rowp, colp, n_parts, nch):
    per_sub = nch * _CHUNK
    total = n_parts * 16 * per_sub
    pad = total - rowp.shape[0]
    rowp = jnp.concatenate([rowp, jnp.full((pad,), _N, jnp.int32)])
    colp = jnp.concatenate([colp, jnp.full((pad,), _TRASH, jnp.int32)])
    shape = (n_parts, 16, nch, _CHUNK)
    return rowp.reshape(shape), colp.reshape(shape)


def _halve(colp, h):
    base = h * _HALF
    inh = (colp >= base) & (colp < base + _HALF)
    return jnp.where(inh, colp - base, _TRASH).astype(jnp.int32)


def _split_cores(rowp, colp, nch):
    half = rowp.shape[0] // 2
    r0, c0 = _chunked(rowp[:half], colp[:half], 1, nch)
    r1, c1 = _chunked(rowp[half:], colp[half:], 1, nch)
    return (jnp.concatenate([r0, r1], axis=0),
            jnp.concatenate([c0, c1], axis=0))


def _edge_prep(row, col):
    rowp = jnp.where(row == col, _N, row).astype(jnp.int32)
    colp = col.astype(jnp.int32)
    ea, eb, ed = [], [], []
    for h in (0, 1):
        ch = _halve(colp, h)
        ra, ca = _chunked(rowp, ch, 1, _NCH_A)
        ea.append((jnp.concatenate([ra, ra], axis=0),
                   jnp.concatenate([ca, ca], axis=0)))
        eb.append(_split_cores(rowp, ch, _NCH_B))
        ed.append(_split_cores(rowp, _halve(row.astype(jnp.int32), h),
                               _NCH_B))
    return ea, eb, ed


def _run_halves(flavor, zts, eh, zeros, combine):
    outs = []
    for h in (0, 1):
        acc = _get_prop(flavor)(zts, eh[h][0], eh[h][1], zeros)
        outs.append(combine(acc)[:_HALF])
    return jnp.concatenate(outs, axis=0)


def _dis_vec(ed, zeros, n):
    ones = jnp.pad(jnp.ones((n, 1), jnp.float32),
                   ((0, _NPAD - n), (0, _CW - 1)))[None]
    deg = _run_halves("B", ones, ed, zeros, lambda a: a[0] + a[1])[:, 0]
    dis = jnp.where(deg > 0, 1.0 / jnp.sqrt(jnp.where(deg > 0, deg, 1.0)), 0.0)
    return dis[:, None]


def _sc_prop(zt, ea, eb, zeros):
    c = zt.shape[1]
    if c == 2 * _CW:
        zts = jnp.stack([zt[:, :_CW], zt[:, _CW:]], axis=0)
        return _run_halves(
            "A", zts, ea, zeros,
            lambda a: jnp.concatenate([a[0], a[1]], axis=1))
    ztp = jnp.pad(zt, ((0, 0), (0, _CW - c)))[None]
    return _run_halves("B", ztp, eb, zeros,
                       lambda a: a[0] + a[1])[:, :c]


def _cheb_conv(h, dis, ea, eb, zeros, W, b):
    Tx0 = h
    out = Tx0 @ W[0]
    acc = _sc_prop(dis * Tx0, ea, eb, zeros)
    Tx1 = -dis * acc
    out = out + Tx1 @ W[1]
    for k in range(2, W.shape[0]):
        acc = _sc_prop(dis * Tx1, ea, eb, zeros)
        Tx2 = -2.0 * dis * acc - Tx0
        out = out + Tx2 @ W[k]
        Tx0, Tx1 = Tx1, Tx2
    return out + b


def _branch(x, row, col, n, params):
    ea, eb, ed = _edge_prep(row, col)
    zeros = jnp.zeros((_ACC_ROWS, _CW), jnp.float32)
    dis = _dis_vec(ed, zeros, n)
    h = jnp.pad(x, ((0, _NPAD - n), (0, 0)))
    for i, (W, b) in enumerate(params):
        h = _cheb_conv(h, dis, ea, eb, zeros, W, b)
        if i < len(params) - 1:
            h = jax.nn.relu(h)
    return h[:n]


def kernel(x, edge_index, params1, params2, params3):
    n1 = _N
    n3 = edge_index.shape[1] // 3
    e1 = edge_index[:, 0:n3]
    e2 = edge_index[:, n3:2 * n3]
    e3 = edge_index[:, 2 * n3:]
    x1 = x[0:2 * n1:2, :]
    x2 = x[1:2 * n1:2, :]
    x3 = x[2 * n1:, :]
    o1 = _branch(x1, e1[0], e1[1], n1, params1)
    o2 = _branch(x2, e2[0], e2[1], n1, params2)
    o3 = _branch(x3, e3[0], e3[1], x3.shape[0], params3)
    uv = jnp.stack([o1, o2], axis=1).reshape(2 * n1, o1.shape[1])
    return jnp.concatenate([uv, o3], axis=0)
